# Initial kernel scaffold; baseline (speedup 1.0000x reference)
import jax
import jax.numpy as jnp
from jax.experimental import pallas as pl


def kernel(user_ids, item_ids, P, Q, U, V, W1, b1, W2, b2, W3, b3, Wo):
    raise NotImplementedError("write your pallas kernel here")


# speedup vs baseline: 2.4837x; 2.4837x over previous
import functools

import jax
import jax.numpy as jnp
from jax import lax
from jax.experimental import pallas as pl
from jax.experimental.pallas import tpu as pltpu
from jax.experimental.pallas import tpu_sc as plsc

NUM_FACTORS = 128
BATCH = 16384

_SC_INFO = plsc.get_sparse_core_info()
_NC = _SC_INFO.num_cores
_NS = _SC_INFO.num_subcores
_NW = _NC * _NS
_B_PER_W = BATCH // _NW
_CHUNK = 128
_NCHUNKS = _B_PER_W // _CHUNK


def _sc_gather_body(uid, iid, p_hbm, q_hbm, u_hbm, v_hbm,
                    op, oq, ou, ov,
                    idx_u, idx_i, bp, bq, bu, bv, sem):
    wid = lax.axis_index("s") * _NC + lax.axis_index("c")
    base = wid * _B_PER_W
    for c in range(_NCHUNKS):
        row0 = base + c * _CHUNK
        pltpu.sync_copy(uid.at[pl.ds(row0, _CHUNK)], idx_u)
        pltpu.sync_copy(iid.at[pl.ds(row0, _CHUNK)], idx_i)
        cp = pltpu.async_copy(p_hbm.at[idx_u], bp, sem)
        cq = pltpu.async_copy(q_hbm.at[idx_i], bq, sem)
        cu = pltpu.async_copy(u_hbm.at[idx_u], bu, sem)
        cv = pltpu.async_copy(v_hbm.at[idx_i], bv, sem)
        cp.wait()
        cq.wait()
        cu.wait()
        cv.wait()
        pltpu.sync_copy(bp, op.at[pl.ds(row0, _CHUNK)])
        pltpu.sync_copy(bq, oq.at[pl.ds(row0, _CHUNK)])
        pltpu.sync_copy(bu, ou.at[pl.ds(row0, _CHUNK)])
        pltpu.sync_copy(bv, ov.at[pl.ds(row0, _CHUNK)])


_ROW_SHAPE = jax.ShapeDtypeStruct((BATCH, NUM_FACTORS), jnp.float32)

_sc_gather = functools.partial(
    pl.kernel,
    mesh=plsc.VectorSubcoreMesh(core_axis_name="c", subcore_axis_name="s"),
    out_type=(_ROW_SHAPE, _ROW_SHAPE, _ROW_SHAPE, _ROW_SHAPE),
    scratch_types=[
        pltpu.VMEM((_CHUNK,), jnp.int32),
        pltpu.VMEM((_CHUNK,), jnp.int32),
        pltpu.VMEM((_CHUNK, NUM_FACTORS), jnp.float32),
        pltpu.VMEM((_CHUNK, NUM_FACTORS), jnp.float32),
        pltpu.VMEM((_CHUNK, NUM_FACTORS), jnp.float32),
        pltpu.VMEM((_CHUNK, NUM_FACTORS), jnp.float32),
        pltpu.SemaphoreType.DMA,
    ],
)(_sc_gather_body)


_R = 2048


def _tc_mlp_body(pm, qm, um, vm, w1a, w1b, b1, w2, b2, w3, b3, wog, woh, out):
    xu = um[...].astype(jnp.bfloat16)
    xv = vm[...].astype(jnp.bfloat16)
    h1 = jnp.dot(xu, w1a[...], preferred_element_type=jnp.float32)
    h1 += jnp.dot(xv, w1b[...], preferred_element_type=jnp.float32)
    h1 = jnp.maximum(h1 + b1[...], 0.0).astype(jnp.bfloat16)
    h2 = jnp.dot(h1, w2[...], preferred_element_type=jnp.float32)
    h2 = jnp.maximum(h2 + b2[...], 0.0).astype(jnp.bfloat16)
    h3 = jnp.dot(h2, w3[...], preferred_element_type=jnp.float32)
    h3 = jnp.maximum(h3 + b3[...], 0.0)
    gmf = pm[...] * qm[...]
    z = jnp.sum(gmf * wog[...], axis=1, keepdims=True)
    z += jnp.sum(h3 * woh[...], axis=1, keepdims=True)
    out[...] = jax.nn.sigmoid(z)


def _tc_mlp(pm, qm, um, vm, w1a, w1b, b1, w2, b2, w3, b3, wog, woh):
    grid = (BATCH // _R,)
    row_spec = pl.BlockSpec((_R, NUM_FACTORS), lambda i: (i, 0))
    full = lambda s: pl.BlockSpec(s, lambda i: (0,) * len(s))
    return pl.pallas_call(
        _tc_mlp_body,
        grid=grid,
        in_specs=[
            row_spec, row_spec, row_spec, row_spec,
            full(w1a.shape), full(w1b.shape), full(b1.shape),
            full(w2.shape), full(b2.shape),
            full(w3.shape), full(b3.shape),
            full(wog.shape), full(woh.shape),
        ],
        out_specs=pl.BlockSpec((_R, 1), lambda i: (i, 0)),
        out_shape=jax.ShapeDtypeStruct((BATCH, 1), jnp.float32),
    )(pm, qm, um, vm, w1a, w1b, b1, w2, b2, w3, b3, wog, woh)


def kernel(user_ids, item_ids, P, Q, U, V, W1, b1, W2, b2, W3, b3, Wo):
    p_mf, q_mf, p_mlp, q_mlp = _sc_gather(user_ids, item_ids, P, Q, U, V)
    bf = jnp.bfloat16
    w1a = W1[:NUM_FACTORS].astype(bf)
    w1b = W1[NUM_FACTORS:].astype(bf)
    wog = Wo[:NUM_FACTORS].reshape(1, NUM_FACTORS)
    woh = Wo[NUM_FACTORS:].reshape(1, -1)
    return _tc_mlp(
        p_mf, q_mf, p_mlp, q_mlp,
        w1a, w1b, b1.reshape(1, -1),
        W2.astype(bf), b2.reshape(1, -1),
        W3.astype(bf), b3.reshape(1, -1),
        wog, woh,
    )
